# Initial kernel scaffold; baseline (speedup 1.0000x reference)
import jax
import jax.numpy as jnp
from jax.experimental import pallas as pl


def kernel(pred, gt, mask):
    raise NotImplementedError("write your pallas kernel here")


# speedup vs baseline: 25.6252x; 25.6252x over previous
import functools

import jax
import jax.numpy as jnp
from jax import lax
from jax.experimental import pallas as pl
from jax.experimental.pallas import tpu as pltpu
from jax.experimental.pallas import tpu_sc as plsc

_EPS = 1e-06
_NEG_RATIO = 3.0

_N = 32 * 512 * 512
_ROWS = 8192
_COLS = 1024
_GRID_A = 32
_BLK_A = _ROWS // _GRID_A

_NW = 32
_SHARD = _N // _NW
_CHUNK = 16384
_NCHUNK = _SHARD // _CHUNK
_LANES = 16

_BTAB = 2176
_ENT = _BTAB * _LANES
_ER = _ENT // 128


def _bce_body(p_ref, g_ref, m_ref, nl_ref, s_ref):
    i = pl.program_id(0)
    p = p_ref[...]
    g = g_ref[...]
    m = m_ref[...]
    log_p = jnp.maximum(jnp.log(p), -100.0)
    log_1mp = jnp.maximum(jnp.log(1.0 - p), -100.0)
    loss = -(g * log_p + (1.0 - g) * log_1mp)
    pos_w = g * m
    neg_w = (1.0 - g) * m
    nl_ref[...] = jnp.maximum(loss * neg_w, 0.0)

    @pl.when(i == 0)
    def _init():
        s_ref[0] = 0.0
        s_ref[1] = 0.0
        s_ref[2] = 0.0

    s_ref[0] += jnp.sum(loss * pos_w)
    s_ref[1] += jnp.sum(pos_w)
    s_ref[2] += jnp.sum(neg_w)


def _pass_a(p2, g2, m2):
    blk = pl.BlockSpec((_BLK_A, _COLS), lambda i: (i, 0))
    return pl.pallas_call(
        _bce_body,
        grid=(_GRID_A,),
        in_specs=[blk, blk, blk],
        out_specs=[
            pl.BlockSpec((_BLK_A, _COLS), lambda i: (i, 0)),
            pl.BlockSpec(memory_space=pltpu.SMEM),
        ],
        out_shape=[
            jax.ShapeDtypeStruct((_ROWS, _COLS), jnp.float32),
            jax.ShapeDtypeStruct((3,), jnp.float32),
        ],
    )(p2, g2, m2)


def _sc_hist_body(nl_hbm, cnt_hbm, sum_hbm, buf, cnt_v, sum_v):
    cid = lax.axis_index("c")
    sid = lax.axis_index("s")
    wid = sid * 2 + cid
    base = wid * _SHARD

    zeros = jnp.zeros((_LANES,), jnp.float32)
    ones = jnp.ones((_LANES,), jnp.float32)
    iota = lax.iota(jnp.int32, _LANES)

    def _zero(i, carry):
        cnt_v[pl.ds(i * _LANES, _LANES)] = zeros
        sum_v[pl.ds(i * _LANES, _LANES)] = zeros
        return carry

    lax.fori_loop(0, _ENT // _LANES, _zero, 0)

    def _one_vreg(off):
        v = buf[pl.ds(off, _LANES)]
        bits = lax.bitcast_convert_type(v, jnp.int32)
        ent = jnp.bitwise_and(lax.shift_right_logical(bits, 15), -16)
        ent = jnp.minimum(ent, _ENT - _LANES) + iota
        plsc.addupdate_scatter(cnt_v, [ent], ones)
        plsc.addupdate_scatter(sum_v, [ent], v)

    def _quad(j, carry):
        off = j * (4 * _LANES)
        _one_vreg(off)
        _one_vreg(off + _LANES)
        _one_vreg(off + 2 * _LANES)
        _one_vreg(off + 3 * _LANES)
        return carry

    for ch in range(_NCHUNK):
        pltpu.sync_copy(nl_hbm.at[pl.ds(base + ch * _CHUNK, _CHUNK)], buf)
        lax.fori_loop(0, _CHUNK // (4 * _LANES), _quad, 0)

    pltpu.sync_copy(cnt_v, cnt_hbm.at[wid])
    pltpu.sync_copy(sum_v, sum_hbm.at[wid])


@functools.cache
def _get_sc_hist():
    return pl.kernel(
        _sc_hist_body,
        out_type=[
            jax.ShapeDtypeStruct((_NW, _ENT), jnp.float32),
            jax.ShapeDtypeStruct((_NW, _ENT), jnp.float32),
        ],
        mesh=plsc.VectorSubcoreMesh(core_axis_name="c", subcore_axis_name="s"),
        compiler_params=pltpu.CompilerParams(needs_layout_passes=False),
        scratch_types=[
            pltpu.VMEM((_CHUNK,), jnp.float32),
            pltpu.VMEM((_ENT,), jnp.float32),
            pltpu.VMEM((_ENT,), jnp.float32),
        ],
    )


def _select_body(cnt_ref, sum_ref, s_ref, out_ref):
    cnt = jnp.sum(cnt_ref[...], axis=0)
    sm = jnp.sum(sum_ref[...], axis=0)
    ent = (lax.broadcasted_iota(jnp.int32, (_ER, 128), 0) * 128
           + lax.broadcasted_iota(jnp.int32, (_ER, 128), 1))

    pos_loss = s_ref[0]
    pos_count = jnp.floor(s_ref[1])
    neg_total = jnp.floor(s_ref[2])
    k = jnp.where(pos_count == 0.0,
                  jnp.minimum(neg_total, 0.0),
                  jnp.minimum(neg_total, pos_count * _NEG_RATIO))

    def count_from(e):
        return jnp.sum(jnp.where(ent >= e, cnt, 0.0))

    def sum_from(e):
        return jnp.sum(jnp.where(ent >= e, sm, 0.0))

    def bs(_, lohi):
        lo, hi = lohi
        mid = (lo + hi) // 2
        pred = count_from(_LANES * (mid + 1)) < k
        return jnp.where(pred, lo, mid + 1), jnp.where(pred, mid, hi)

    lo, hi = lax.fori_loop(
        0, 12, bs, (jnp.int32(0), jnp.int32(_BTAB - 1)))
    bstar = hi

    c_above = count_from(_LANES * (bstar + 1))
    s_above = sum_from(_LANES * (bstar + 1))
    c_b = count_from(_LANES * bstar) - c_above
    s_b = sum_from(_LANES * bstar) - s_above
    m = jnp.clip(k - c_above, 0.0, c_b)
    neg_sum = jnp.where(
        k > 0.0, s_above + m * (s_b / jnp.maximum(c_b, 1.0)), 0.0)
    out_ref[0] = (pos_loss + neg_sum) / (pos_count + k + _EPS)


def _pass_c(cnt3, sum3, scal):
    return pl.pallas_call(
        _select_body,
        in_specs=[
            pl.BlockSpec(memory_space=pltpu.VMEM),
            pl.BlockSpec(memory_space=pltpu.VMEM),
            pl.BlockSpec(memory_space=pltpu.SMEM),
        ],
        out_specs=pl.BlockSpec(memory_space=pltpu.SMEM),
        out_shape=jax.ShapeDtypeStruct((1,), jnp.float32),
    )(cnt3, sum3, scal)


def kernel(pred, gt, mask):
    p2 = pred.reshape(_ROWS, _COLS)
    g2 = gt.reshape(_ROWS, _COLS)
    m2 = mask.reshape(_ROWS, _COLS)
    nl, scal = _pass_a(p2, g2, m2)
    cnt, sm = _get_sc_hist()(nl.reshape(_N))
    out = _pass_c(cnt.reshape(_NW, _ER, 128), sm.reshape(_NW, _ER, 128), scal)
    return out.reshape(())


# speedup vs baseline: 26.8207x; 1.0467x over previous
import functools

import jax
import jax.numpy as jnp
from jax import lax
from jax.experimental import pallas as pl
from jax.experimental.pallas import tpu as pltpu
from jax.experimental.pallas import tpu_sc as plsc

_EPS = 1e-06
_NEG_RATIO = 3.0

_N = 32 * 512 * 512
_ROWS = 8192
_COLS = 1024
_GRID_A = 32
_BLK_A = _ROWS // _GRID_A

_NW = 32
_SHARD = _N // _NW
_CHUNK = 16384
_NCHUNK = _SHARD // _CHUNK
_LANES = 16

_BTAB = 2176
_ENT = _BTAB * _LANES
_ER = _ENT // 128


def _bce_body(p_ref, g_ref, m_ref, nl_ref, s_ref):
    i = pl.program_id(0)
    p = p_ref[...]
    g = g_ref[...]
    m = m_ref[...]
    log_p = jnp.maximum(jnp.log(p), -100.0)
    log_1mp = jnp.maximum(jnp.log(1.0 - p), -100.0)
    loss = -(g * log_p + (1.0 - g) * log_1mp)
    pos_w = g * m
    neg_w = (1.0 - g) * m
    nl_ref[...] = jnp.maximum(loss * neg_w, 0.0)

    @pl.when(i == 0)
    def _init():
        s_ref[0] = 0.0
        s_ref[1] = 0.0
        s_ref[2] = 0.0

    s_ref[0] += jnp.sum(loss * pos_w)
    s_ref[1] += jnp.sum(pos_w)
    s_ref[2] += jnp.sum(neg_w)


def _pass_a(p2, g2, m2):
    blk = pl.BlockSpec((_BLK_A, _COLS), lambda i: (i, 0))
    return pl.pallas_call(
        _bce_body,
        grid=(_GRID_A,),
        in_specs=[blk, blk, blk],
        out_specs=[
            pl.BlockSpec((_BLK_A, _COLS), lambda i: (i, 0)),
            pl.BlockSpec(memory_space=pltpu.SMEM),
        ],
        out_shape=[
            jax.ShapeDtypeStruct((_ROWS, _COLS), jnp.float32),
            jax.ShapeDtypeStruct((3,), jnp.float32),
        ],
    )(p2, g2, m2)


_UNROLL = 16


def _sc_hist_body(nl_hbm, cnt_hbm, sum_hbm, buf0, buf1, cnt_v, sum_v,
                  sem0, sem1):
    cid = lax.axis_index("c")
    sid = lax.axis_index("s")
    wid = sid * 2 + cid
    base = wid * _SHARD

    zeros = jnp.zeros((_LANES,), jnp.float32)
    ones = jnp.ones((_LANES,), jnp.float32)
    iota = lax.iota(jnp.int32, _LANES)

    def _zero(i, carry):
        cnt_v[pl.ds(i * _LANES, _LANES)] = zeros
        sum_v[pl.ds(i * _LANES, _LANES)] = zeros
        return carry

    lax.fori_loop(0, _ENT // _LANES, _zero, 0)

    def _one_vreg(buf, off):
        v = buf[pl.ds(off, _LANES)]
        bits = lax.bitcast_convert_type(v, jnp.int32)
        ent = jnp.bitwise_and(lax.shift_right_logical(bits, 15), -16)
        ent = jnp.minimum(ent, _ENT - _LANES) + iota
        plsc.addupdate_scatter(cnt_v, [ent], ones)
        plsc.addupdate_scatter(sum_v, [ent], v)

    def _process(buf):
        def _block(j, carry):
            off = j * (_UNROLL * _LANES)
            for u in range(_UNROLL):
                _one_vreg(buf, off + u * _LANES)
            return carry

        lax.fori_loop(0, _CHUNK // (_UNROLL * _LANES), _block, 0)

    bufs = (buf0, buf1)
    sems = (sem0, sem1)
    cp = pltpu.async_copy(nl_hbm.at[pl.ds(base, _CHUNK)], buf0, sem0)
    for ch in range(_NCHUNK):
        nxt = ch + 1
        if nxt < _NCHUNK:
            cp_next = pltpu.async_copy(
                nl_hbm.at[pl.ds(base + nxt * _CHUNK, _CHUNK)],
                bufs[nxt % 2], sems[nxt % 2])
        cp.wait()
        _process(bufs[ch % 2])
        if nxt < _NCHUNK:
            cp = cp_next

    pltpu.sync_copy(cnt_v, cnt_hbm.at[wid])
    pltpu.sync_copy(sum_v, sum_hbm.at[wid])


@functools.cache
def _get_sc_hist():
    return pl.kernel(
        _sc_hist_body,
        out_type=[
            jax.ShapeDtypeStruct((_NW, _ENT), jnp.float32),
            jax.ShapeDtypeStruct((_NW, _ENT), jnp.float32),
        ],
        mesh=plsc.VectorSubcoreMesh(core_axis_name="c", subcore_axis_name="s"),
        compiler_params=pltpu.CompilerParams(needs_layout_passes=False),
        scratch_types=[
            pltpu.VMEM((_CHUNK,), jnp.float32),
            pltpu.VMEM((_CHUNK,), jnp.float32),
            pltpu.VMEM((_ENT,), jnp.float32),
            pltpu.VMEM((_ENT,), jnp.float32),
            pltpu.SemaphoreType.DMA,
            pltpu.SemaphoreType.DMA,
        ],
    )


def _select_body(cnt_ref, sum_ref, s_ref, out_ref):
    cnt = jnp.sum(cnt_ref[...], axis=0)
    sm = jnp.sum(sum_ref[...], axis=0)
    ent = (lax.broadcasted_iota(jnp.int32, (_ER, 128), 0) * 128
           + lax.broadcasted_iota(jnp.int32, (_ER, 128), 1))

    pos_loss = s_ref[0]
    pos_count = jnp.floor(s_ref[1])
    neg_total = jnp.floor(s_ref[2])
    k = jnp.where(pos_count == 0.0,
                  jnp.minimum(neg_total, 0.0),
                  jnp.minimum(neg_total, pos_count * _NEG_RATIO))

    def count_from(e):
        return jnp.sum(jnp.where(ent >= e, cnt, 0.0))

    def sum_from(e):
        return jnp.sum(jnp.where(ent >= e, sm, 0.0))

    def bs(_, lohi):
        lo, hi = lohi
        mid = (lo + hi) // 2
        pred = count_from(_LANES * (mid + 1)) < k
        return jnp.where(pred, lo, mid + 1), jnp.where(pred, mid, hi)

    lo, hi = lax.fori_loop(
        0, 12, bs, (jnp.int32(0), jnp.int32(_BTAB - 1)))
    bstar = hi

    c_above = count_from(_LANES * (bstar + 1))
    s_above = sum_from(_LANES * (bstar + 1))
    c_b = count_from(_LANES * bstar) - c_above
    s_b = sum_from(_LANES * bstar) - s_above
    m = jnp.clip(k - c_above, 0.0, c_b)
    neg_sum = jnp.where(
        k > 0.0, s_above + m * (s_b / jnp.maximum(c_b, 1.0)), 0.0)
    out_ref[0] = (pos_loss + neg_sum) / (pos_count + k + _EPS)


def _pass_c(cnt3, sum3, scal):
    return pl.pallas_call(
        _select_body,
        in_specs=[
            pl.BlockSpec(memory_space=pltpu.VMEM),
            pl.BlockSpec(memory_space=pltpu.VMEM),
            pl.BlockSpec(memory_space=pltpu.SMEM),
        ],
        out_specs=pl.BlockSpec(memory_space=pltpu.SMEM),
        out_shape=jax.ShapeDtypeStruct((1,), jnp.float32),
    )(cnt3, sum3, scal)


def kernel(pred, gt, mask):
    p2 = pred.reshape(_ROWS, _COLS)
    g2 = gt.reshape(_ROWS, _COLS)
    m2 = mask.reshape(_ROWS, _COLS)
    nl, scal = _pass_a(p2, g2, m2)
    cnt, sm = _get_sc_hist()(nl.reshape(_N))
    out = _pass_c(cnt.reshape(_NW, _ER, 128), sm.reshape(_NW, _ER, 128), scal)
    return out.reshape(())


# speedup vs baseline: 38.7923x; 1.4464x over previous
import functools

import jax
import jax.numpy as jnp
from jax import lax
from jax.experimental import pallas as pl
from jax.experimental.pallas import tpu as pltpu
from jax.experimental.pallas import tpu_sc as plsc

_EPS = 1e-06
_NEG_RATIO = 3.0

_N = 32 * 512 * 512
_ROWS = 8192
_COLS = 1024
_GRID_A = 32
_BLK_A = _ROWS // _GRID_A

_NW = 32
_SHARD = _N // _NW
_CHUNK = 16384
_NCHUNK = _SHARD // _CHUNK
_LANES = 16

_BTAB = 2176
_ENT = _BTAB * _LANES
_ER = _ENT // 128


def _bce_body(p_ref, g_ref, m_ref, nl_ref, s_ref):
    i = pl.program_id(0)
    p = p_ref[...]
    g = g_ref[...]
    m = m_ref[...]
    log_p = jnp.maximum(jnp.log(p), -100.0)
    log_1mp = jnp.maximum(jnp.log(1.0 - p), -100.0)
    loss = -(g * log_p + (1.0 - g) * log_1mp)
    pos_w = g * m
    neg_w = (1.0 - g) * m
    nl_ref[...] = jnp.minimum(jnp.maximum(loss * neg_w, 0.0), 100.0)

    @pl.when(i == 0)
    def _init():
        s_ref[0] = 0.0
        s_ref[1] = 0.0
        s_ref[2] = 0.0

    s_ref[0] += jnp.sum(loss * pos_w)
    s_ref[1] += jnp.sum(pos_w)
    s_ref[2] += jnp.sum(neg_w)


def _pass_a(p2, g2, m2):
    blk = pl.BlockSpec((_BLK_A, _COLS), lambda i: (i, 0))
    return pl.pallas_call(
        _bce_body,
        grid=(_GRID_A,),
        in_specs=[blk, blk, blk],
        out_specs=[
            pl.BlockSpec((_BLK_A, _COLS), lambda i: (i, 0)),
            pl.BlockSpec(memory_space=pltpu.SMEM),
        ],
        out_shape=[
            jax.ShapeDtypeStruct((_ROWS, _COLS), jnp.float32),
            jax.ShapeDtypeStruct((3,), jnp.float32),
        ],
    )(p2, g2, m2)


_UNROLL = 16


def _sc_hist_body(nl_hbm, cnt_hbm, sum_hbm, buf0, buf1, cnt_v, sum_v,
                  sem0, sem1):
    cid = lax.axis_index("c")
    sid = lax.axis_index("s")
    wid = sid * 2 + cid
    base = wid * _SHARD

    zeros = jnp.zeros((_LANES,), jnp.float32)
    ones = jnp.ones((_LANES,), jnp.float32)
    iota = lax.iota(jnp.int32, _LANES)

    def _zero(i, carry):
        base0 = i * (8 * _LANES)
        for u in range(8):
            cnt_v[pl.ds(base0 + u * _LANES, _LANES)] = zeros
            sum_v[pl.ds(base0 + u * _LANES, _LANES)] = zeros
        return carry

    lax.fori_loop(0, _ENT // (8 * _LANES), _zero, 0)

    def _process(buf):
        def _block(j, carry):
            off = j * (_UNROLL * _LANES)
            vs = [buf[pl.ds(off + u * _LANES, _LANES)]
                  for u in range(_UNROLL)]
            ents = []
            for v in vs:
                bits = lax.bitcast_convert_type(v, jnp.int32)
                ent = jnp.bitwise_and(
                    lax.shift_right_logical(bits, 15), -16)
                ents.append(jnp.bitwise_or(ent, iota))
            for v, ent in zip(vs, ents):
                plsc.addupdate_scatter(cnt_v, [ent], ones)
                plsc.addupdate_scatter(sum_v, [ent], v)
            return carry

        lax.fori_loop(0, _CHUNK // (_UNROLL * _LANES), _block, 0)

    bufs = (buf0, buf1)
    sems = (sem0, sem1)
    cp = pltpu.async_copy(nl_hbm.at[pl.ds(base, _CHUNK)], buf0, sem0)
    for ch in range(_NCHUNK):
        nxt = ch + 1
        if nxt < _NCHUNK:
            cp_next = pltpu.async_copy(
                nl_hbm.at[pl.ds(base + nxt * _CHUNK, _CHUNK)],
                bufs[nxt % 2], sems[nxt % 2])
        cp.wait()
        _process(bufs[ch % 2])
        if nxt < _NCHUNK:
            cp = cp_next

    pltpu.sync_copy(cnt_v, cnt_hbm.at[wid])
    pltpu.sync_copy(sum_v, sum_hbm.at[wid])


@functools.cache
def _get_sc_hist():
    return pl.kernel(
        _sc_hist_body,
        out_type=[
            jax.ShapeDtypeStruct((_NW, _ENT), jnp.float32),
            jax.ShapeDtypeStruct((_NW, _ENT), jnp.float32),
        ],
        mesh=plsc.VectorSubcoreMesh(core_axis_name="c", subcore_axis_name="s"),
        compiler_params=pltpu.CompilerParams(needs_layout_passes=False),
        scratch_types=[
            pltpu.VMEM((_CHUNK,), jnp.float32),
            pltpu.VMEM((_CHUNK,), jnp.float32),
            pltpu.VMEM((_ENT,), jnp.float32),
            pltpu.VMEM((_ENT,), jnp.float32),
            pltpu.SemaphoreType.DMA,
            pltpu.SemaphoreType.DMA,
        ],
    )


def _select_body(cnt_ref, sum_ref, s_ref, out_ref):
    cnt = jnp.sum(cnt_ref[...], axis=0)
    sm = jnp.sum(sum_ref[...], axis=0)
    ent = (lax.broadcasted_iota(jnp.int32, (_ER, 128), 0) * 128
           + lax.broadcasted_iota(jnp.int32, (_ER, 128), 1))

    pos_loss = s_ref[0]
    pos_count = jnp.floor(s_ref[1])
    neg_total = jnp.floor(s_ref[2])
    k = jnp.where(pos_count == 0.0,
                  jnp.minimum(neg_total, 0.0),
                  jnp.minimum(neg_total, pos_count * _NEG_RATIO))

    def count_from(e):
        return jnp.sum(jnp.where(ent >= e, cnt, 0.0))

    def sum_from(e):
        return jnp.sum(jnp.where(ent >= e, sm, 0.0))

    def bs(_, lohi):
        lo, hi = lohi
        mid = (lo + hi) // 2
        pred = count_from(_LANES * (mid + 1)) < k
        return jnp.where(pred, lo, mid + 1), jnp.where(pred, mid, hi)

    lo, hi = lax.fori_loop(
        0, 12, bs, (jnp.int32(0), jnp.int32(_BTAB - 1)))
    bstar = hi

    c_above = count_from(_LANES * (bstar + 1))
    s_above = sum_from(_LANES * (bstar + 1))
    c_b = count_from(_LANES * bstar) - c_above
    s_b = sum_from(_LANES * bstar) - s_above
    m = jnp.clip(k - c_above, 0.0, c_b)
    neg_sum = jnp.where(
        k > 0.0, s_above + m * (s_b / jnp.maximum(c_b, 1.0)), 0.0)
    out_ref[0] = (pos_loss + neg_sum) / (pos_count + k + _EPS)


def _pass_c(cnt3, sum3, scal):
    return pl.pallas_call(
        _select_body,
        in_specs=[
            pl.BlockSpec(memory_space=pltpu.VMEM),
            pl.BlockSpec(memory_space=pltpu.VMEM),
            pl.BlockSpec(memory_space=pltpu.SMEM),
        ],
        out_specs=pl.BlockSpec(memory_space=pltpu.SMEM),
        out_shape=jax.ShapeDtypeStruct((1,), jnp.float32),
    )(cnt3, sum3, scal)


def kernel(pred, gt, mask):
    p2 = pred.reshape(_ROWS, _COLS)
    g2 = gt.reshape(_ROWS, _COLS)
    m2 = mask.reshape(_ROWS, _COLS)
    nl, scal = _pass_a(p2, g2, m2)
    cnt, sm = _get_sc_hist()(nl.reshape(_N))
    out = _pass_c(cnt.reshape(_NW, _ER, 128), sm.reshape(_NW, _ER, 128), scal)
    return out.reshape(())


# speedup vs baseline: 43.7316x; 1.1273x over previous
import functools

import jax
import jax.numpy as jnp
from jax import lax
from jax.experimental import pallas as pl
from jax.experimental.pallas import tpu as pltpu
from jax.experimental.pallas import tpu_sc as plsc

_EPS = 1e-06
_NEG_RATIO = 3.0

_N = 32 * 512 * 512
_ROWS = 8192
_COLS = 1024
_GRID_A = 32
_BLK_A = _ROWS // _GRID_A

_NW = 32
_SHARD_ROWS = _ROWS // _NW
_CHUNK_ROWS = 16
_CHUNK = _CHUNK_ROWS * _COLS
_NCHUNK = _SHARD_ROWS // _CHUNK_ROWS
_LANES = 16

_BTAB = 2176
_ENT = _BTAB * _LANES
_ER = _ENT // 128


def _bce_body(p_ref, g_ref, m_ref, nl_ref, s_ref):
    i = pl.program_id(0)
    p = p_ref[...]
    g = g_ref[...]
    m = m_ref[...]
    log_p = jnp.maximum(jnp.log(p), -100.0)
    log_1mp = jnp.maximum(jnp.log(1.0 - p), -100.0)
    loss = -(g * log_p + (1.0 - g) * log_1mp)
    pos_w = g * m
    neg_w = (1.0 - g) * m
    nl_ref[...] = jnp.minimum(jnp.maximum(loss * neg_w, 0.0), 100.0)

    @pl.when(i == 0)
    def _init():
        s_ref[0] = 0.0
        s_ref[1] = 0.0
        s_ref[2] = 0.0

    s_ref[0] += jnp.sum(loss * pos_w)
    s_ref[1] += jnp.sum(pos_w)
    s_ref[2] += jnp.sum(neg_w)


def _pass_a(p2, g2, m2):
    blk = pl.BlockSpec((_BLK_A, _COLS), lambda i: (i, 0))
    return pl.pallas_call(
        _bce_body,
        grid=(_GRID_A,),
        in_specs=[blk, blk, blk],
        out_specs=[
            pl.BlockSpec((_BLK_A, _COLS), lambda i: (i, 0)),
            pl.BlockSpec(memory_space=pltpu.SMEM),
        ],
        out_shape=[
            jax.ShapeDtypeStruct((_ROWS, _COLS), jnp.float32),
            jax.ShapeDtypeStruct((3,), jnp.float32),
        ],
    )(p2, g2, m2)


_UNROLL = 16


def _sc_hist_body(nl_hbm, cnt_hbm, sum_hbm, buf0, buf1, cnt_v, sum_v,
                  sem0, sem1):
    cid = lax.axis_index("c")
    sid = lax.axis_index("s")
    wid = sid * 2 + cid
    base = wid * _SHARD_ROWS

    zeros = jnp.zeros((_LANES,), jnp.float32)
    ones = jnp.ones((_LANES,), jnp.float32)
    iota = lax.iota(jnp.int32, _LANES)

    def _zero(i, carry):
        base0 = i * (8 * _LANES)
        for u in range(8):
            cnt_v[pl.ds(base0 + u * _LANES, _LANES)] = zeros
            sum_v[pl.ds(base0 + u * _LANES, _LANES)] = zeros
        return carry

    lax.fori_loop(0, _ENT // (8 * _LANES), _zero, 0)

    def _process(buf):
        def _block(j, carry):
            row = j // 4
            col0 = (j % 4) * (_UNROLL * _LANES)
            vs = [buf[row, pl.ds(col0 + u * _LANES, _LANES)]
                  for u in range(_UNROLL)]
            ents = []
            for v in vs:
                bits = lax.bitcast_convert_type(v, jnp.int32)
                ent = jnp.bitwise_and(
                    lax.shift_right_logical(bits, 15), -16)
                ents.append(jnp.bitwise_or(ent, iota))
            for v, ent in zip(vs, ents):
                plsc.addupdate_scatter(cnt_v, [ent], ones)
                plsc.addupdate_scatter(sum_v, [ent], v)
            return carry

        lax.fori_loop(0, _CHUNK // (_UNROLL * _LANES), _block, 0)

    bufs = (buf0, buf1)
    sems = (sem0, sem1)
    cp = pltpu.async_copy(
        nl_hbm.at[pl.ds(base, _CHUNK_ROWS)], buf0, sem0)
    for ch in range(_NCHUNK):
        nxt = ch + 1
        if nxt < _NCHUNK:
            cp_next = pltpu.async_copy(
                nl_hbm.at[pl.ds(base + nxt * _CHUNK_ROWS, _CHUNK_ROWS)],
                bufs[nxt % 2], sems[nxt % 2])
        cp.wait()
        _process(bufs[ch % 2])
        if nxt < _NCHUNK:
            cp = cp_next

    pltpu.sync_copy(cnt_v, cnt_hbm.at[wid])
    pltpu.sync_copy(sum_v, sum_hbm.at[wid])


@functools.cache
def _get_sc_hist():
    return pl.kernel(
        _sc_hist_body,
        out_type=[
            jax.ShapeDtypeStruct((_NW, _ENT), jnp.float32),
            jax.ShapeDtypeStruct((_NW, _ENT), jnp.float32),
        ],
        mesh=plsc.VectorSubcoreMesh(core_axis_name="c", subcore_axis_name="s"),
        compiler_params=pltpu.CompilerParams(
            needs_layout_passes=False, use_tc_tiling_on_sc=True),
        scratch_types=[
            pltpu.VMEM((_CHUNK_ROWS, _COLS), jnp.float32),
            pltpu.VMEM((_CHUNK_ROWS, _COLS), jnp.float32),
            pltpu.VMEM((_ENT,), jnp.float32),
            pltpu.VMEM((_ENT,), jnp.float32),
            pltpu.SemaphoreType.DMA,
            pltpu.SemaphoreType.DMA,
        ],
    )


def _select_body(cnt_ref, sum_ref, s_ref, out_ref):
    cnt = jnp.sum(cnt_ref[...], axis=0)
    sm = jnp.sum(sum_ref[...], axis=0)
    ent = (lax.broadcasted_iota(jnp.int32, (_ER, 128), 0) * 128
           + lax.broadcasted_iota(jnp.int32, (_ER, 128), 1))

    pos_loss = s_ref[0]
    pos_count = jnp.floor(s_ref[1])
    neg_total = jnp.floor(s_ref[2])
    k = jnp.where(pos_count == 0.0,
                  jnp.minimum(neg_total, 0.0),
                  jnp.minimum(neg_total, pos_count * _NEG_RATIO))

    def count_from(e):
        return jnp.sum(jnp.where(ent >= e, cnt, 0.0))

    def sum_from(e):
        return jnp.sum(jnp.where(ent >= e, sm, 0.0))

    def bs(_, lohi):
        lo, hi = lohi
        mid = (lo + hi) // 2
        pred = count_from(_LANES * (mid + 1)) < k
        return jnp.where(pred, lo, mid + 1), jnp.where(pred, mid, hi)

    lo, hi = lax.fori_loop(
        0, 12, bs, (jnp.int32(0), jnp.int32(_BTAB - 1)))
    bstar = hi

    c_above = count_from(_LANES * (bstar + 1))
    s_above = sum_from(_LANES * (bstar + 1))
    c_b = count_from(_LANES * bstar) - c_above
    s_b = sum_from(_LANES * bstar) - s_above
    m = jnp.clip(k - c_above, 0.0, c_b)
    neg_sum = jnp.where(
        k > 0.0, s_above + m * (s_b / jnp.maximum(c_b, 1.0)), 0.0)
    out_ref[0] = (pos_loss + neg_sum) / (pos_count + k + _EPS)


def _pass_c(cnt3, sum3, scal):
    return pl.pallas_call(
        _select_body,
        in_specs=[
            pl.BlockSpec(memory_space=pltpu.VMEM),
            pl.BlockSpec(memory_space=pltpu.VMEM),
            pl.BlockSpec(memory_space=pltpu.SMEM),
        ],
        out_specs=pl.BlockSpec(memory_space=pltpu.SMEM),
        out_shape=jax.ShapeDtypeStruct((1,), jnp.float32),
    )(cnt3, sum3, scal)


def kernel(pred, gt, mask):
    p2 = pred.reshape(_ROWS, _COLS)
    g2 = gt.reshape(_ROWS, _COLS)
    m2 = mask.reshape(_ROWS, _COLS)
    nl, scal = _pass_a(p2, g2, m2)
    cnt, sm = _get_sc_hist()(nl)
    out = _pass_c(cnt.reshape(_NW, _ER, 128), sm.reshape(_NW, _ER, 128), scal)
    return out.reshape(())


# speedup vs baseline: 84.9606x; 1.9428x over previous
import functools

import jax
import jax.numpy as jnp
from jax import lax
from jax.experimental import pallas as pl
from jax.experimental.pallas import tpu as pltpu
from jax.experimental.pallas import tpu_sc as plsc

_EPS = 1e-06
_NEG_RATIO = 3.0

_N = 32 * 512 * 512
_D0 = 32
_DR = 512
_DC = 512

_NW = 32
_CHUNK_ROWS = 32
_CHUNK = _CHUNK_ROWS * _DC
_NCHUNK = _DR // _CHUNK_ROWS
_LANES = 16

_BTAB = 2176
_ENT = _BTAB * _LANES


def _bce_body(p_ref, g_ref, m_ref, nl_ref, s_ref):
    i = pl.program_id(0)
    p = p_ref[...]
    g = g_ref[...]
    m = m_ref[...]
    log_p = jnp.maximum(jnp.log(p), -100.0)
    log_1mp = jnp.maximum(jnp.log(1.0 - p), -100.0)
    loss = -(g * log_p + (1.0 - g) * log_1mp)
    pos_w = g * m
    neg_w = (1.0 - g) * m
    nl_ref[...] = jnp.minimum(jnp.maximum(loss * neg_w, 0.0), 100.0)

    @pl.when(i == 0)
    def _init():
        s_ref[0] = 0.0
        s_ref[1] = 0.0
        s_ref[2] = 0.0

    s_ref[0] += jnp.sum(loss * pos_w)
    s_ref[1] += jnp.sum(pos_w)
    s_ref[2] += jnp.sum(neg_w)


def _pass_a(p, g, m):
    blk = pl.BlockSpec((1, _DR, _DC), lambda i: (i, 0, 0))
    return pl.pallas_call(
        _bce_body,
        grid=(_D0,),
        in_specs=[blk, blk, blk],
        out_specs=[
            pl.BlockSpec((1, _DR, _DC), lambda i: (i, 0, 0)),
            pl.BlockSpec(memory_space=pltpu.SMEM),
        ],
        out_shape=[
            jax.ShapeDtypeStruct((_D0, _DR, _DC), jnp.float32),
            jax.ShapeDtypeStruct((3,), jnp.float32),
        ],
    )(p, g, m)


_UNROLL = 16


def _sc_hist_body(nl_hbm, cnt_hbm, sum_hbm, buf0, buf1, cnt_v, sum_v,
                  sem0, sem1):
    cid = lax.axis_index("c")
    sid = lax.axis_index("s")
    wid = sid * 2 + cid

    zeros = jnp.zeros((_LANES,), jnp.float32)
    ones = jnp.ones((_LANES,), jnp.float32)
    iota = lax.iota(jnp.int32, _LANES)

    def _zero(i, carry):
        base0 = i * (8 * _LANES)
        for u in range(8):
            cnt_v[pl.ds(base0 + u * _LANES, _LANES)] = zeros
            sum_v[pl.ds(base0 + u * _LANES, _LANES)] = zeros
        return carry

    lax.fori_loop(0, _ENT // (8 * _LANES), _zero, 0)

    def _process(buf):
        def _block(j, carry):
            row = j // 2
            col0 = (j % 2) * (_UNROLL * _LANES)
            vs = [buf[row, pl.ds(col0 + u * _LANES, _LANES)]
                  for u in range(_UNROLL)]
            ents = []
            for v in vs:
                bits = lax.bitcast_convert_type(v, jnp.int32)
                ent = jnp.bitwise_and(
                    lax.shift_right_logical(bits, 15), -16)
                ents.append(jnp.bitwise_or(ent, iota))
            for v, ent in zip(vs, ents):
                plsc.addupdate_scatter(cnt_v, [ent], ones)
                plsc.addupdate_scatter(sum_v, [ent], v)
            return carry

        lax.fori_loop(0, _CHUNK // (_UNROLL * _LANES), _block, 0)

    bufs = (buf0, buf1)
    sems = (sem0, sem1)
    cp = pltpu.async_copy(
        nl_hbm.at[wid, pl.ds(0, _CHUNK_ROWS)], buf0, sem0)
    for ch in range(_NCHUNK):
        nxt = ch + 1
        if nxt < _NCHUNK:
            cp_next = pltpu.async_copy(
                nl_hbm.at[wid, pl.ds(nxt * _CHUNK_ROWS, _CHUNK_ROWS)],
                bufs[nxt % 2], sems[nxt % 2])
        cp.wait()
        _process(bufs[ch % 2])
        if nxt < _NCHUNK:
            cp = cp_next

    pltpu.sync_copy(cnt_v, cnt_hbm.at[wid])
    pltpu.sync_copy(sum_v, sum_hbm.at[wid])


@functools.cache
def _get_sc_hist():
    return pl.kernel(
        _sc_hist_body,
        out_type=[
            jax.ShapeDtypeStruct((_NW, _ENT), jnp.float32),
            jax.ShapeDtypeStruct((_NW, _ENT), jnp.float32),
        ],
        mesh=plsc.VectorSubcoreMesh(core_axis_name="c", subcore_axis_name="s"),
        compiler_params=pltpu.CompilerParams(
            needs_layout_passes=False, use_tc_tiling_on_sc=True),
        scratch_types=[
            pltpu.VMEM((_CHUNK_ROWS, _DC), jnp.float32),
            pltpu.VMEM((_CHUNK_ROWS, _DC), jnp.float32),
            pltpu.VMEM((_ENT,), jnp.float32),
            pltpu.VMEM((_ENT,), jnp.float32),
            pltpu.SemaphoreType.DMA,
            pltpu.SemaphoreType.DMA,
        ],
    )


def _select_body(cnt_ref, sum_ref, s_ref, out_ref):
    cnt = jnp.sum(cnt_ref[...], axis=0, keepdims=True)
    sm = jnp.sum(sum_ref[...], axis=0, keepdims=True)
    ent = lax.broadcasted_iota(jnp.int32, (1, _ENT), 1)

    pos_loss = s_ref[0]
    pos_count = jnp.floor(s_ref[1])
    neg_total = jnp.floor(s_ref[2])
    k = jnp.where(pos_count == 0.0,
                  jnp.minimum(neg_total, 0.0),
                  jnp.minimum(neg_total, pos_count * _NEG_RATIO))

    def count_from(e):
        return jnp.sum(jnp.where(ent >= e, cnt, 0.0))

    def sum_from(e):
        return jnp.sum(jnp.where(ent >= e, sm, 0.0))

    def bs(_, lohi):
        lo, hi = lohi
        mid = (lo + hi) // 2
        pred = count_from(_LANES * (mid + 1)) < k
        return jnp.where(pred, lo, mid + 1), jnp.where(pred, mid, hi)

    lo, hi = lax.fori_loop(
        0, 12, bs, (jnp.int32(0), jnp.int32(_BTAB - 1)))
    bstar = hi

    c_above = count_from(_LANES * (bstar + 1))
    s_above = sum_from(_LANES * (bstar + 1))
    c_b = count_from(_LANES * bstar) - c_above
    s_b = sum_from(_LANES * bstar) - s_above
    m = jnp.clip(k - c_above, 0.0, c_b)
    neg_sum = jnp.where(
        k > 0.0, s_above + m * (s_b / jnp.maximum(c_b, 1.0)), 0.0)
    out_ref[0] = (pos_loss + neg_sum) / (pos_count + k + _EPS)


def _pass_c(cnt2, sum2, scal):
    return pl.pallas_call(
        _select_body,
        in_specs=[
            pl.BlockSpec(memory_space=pltpu.VMEM),
            pl.BlockSpec(memory_space=pltpu.VMEM),
            pl.BlockSpec(memory_space=pltpu.SMEM),
        ],
        out_specs=pl.BlockSpec(memory_space=pltpu.SMEM),
        out_shape=jax.ShapeDtypeStruct((1,), jnp.float32),
    )(cnt2, sum2, scal)


def kernel(pred, gt, mask):
    nl, scal = _pass_a(pred, gt, mask)
    cnt, sm = _get_sc_hist()(nl)
    out = _pass_c(cnt, sm, scal)
    return out.reshape(())
